# baseline (speedup 1.0000x reference)
import functools

import jax
import jax.numpy as jnp
from jax import lax
from jax.experimental import pallas as pl
from jax.experimental.pallas import tpu as pltpu
from jax.experimental.pallas import tpu_sc as plsc

N_NODES_C = 10000
N_PAD = 10240
N_EDGES_C = 320000
D_C = 128
DH = 64
CHUNK = 80
ROWS_PER_TILE = N_PAD // 16
EDGES_PER_TILE = N_EDGES_C // 16
N_CHUNKS = EDGES_PER_TILE // CHUNK
NBUF = 5
LA = NBUF - 1
ZROWS = 128


def _sc_aggregate(x01, src1d, dst2d):
    mesh = plsc.VectorSubcoreMesh(core_axis_name="c", subcore_axis_name="s")

    @functools.partial(
        pl.kernel,
        out_type=(
            jax.ShapeDtypeStruct((2 * N_PAD, DH), jnp.float32),
            jax.ShapeDtypeStruct((2 * N_PAD, 16), jnp.float32),
        ),
        mesh=mesh,
        compiler_params=pltpu.CompilerParams(use_tc_tiling_on_sc=False),
        scratch_types=[
            pltpu.VMEM_SHARED((N_PAD, DH), jnp.float32),
            pltpu.VMEM_SHARED((N_PAD, 16), jnp.float32),
            pltpu.VMEM((ZROWS, DH), jnp.float32),
            pltpu.VMEM((ZROWS, 16), jnp.float32),
            pltpu.VMEM((EDGES_PER_TILE,), jnp.int32),
            pltpu.VMEM((N_CHUNKS, CHUNK), jnp.int32),
            pltpu.VMEM((NBUF, CHUNK, DH), jnp.float32),
            pltpu.VMEM((CHUNK, 16), jnp.float32),
        ] + [pltpu.SemaphoreType.DMA] * (3 * NBUF),
    )
    def agg_kernel(x_hbm, src_hbm, dst_hbm, agg_out, cnt_out,
                   acc_sh, cnt_sh, zbuf, zbufc, gidx, didx, rows, ones_v,
                   *sems):
        cid = lax.axis_index("c")
        sid = lax.axis_index("s")
        r0 = sid * jnp.int32(ROWS_PER_TILE)
        gsems = sems[0:NBUF]
        ssems = sems[NBUF:2 * NBUF]
        csems = sems[2 * NBUF:3 * NBUF]

        trow = sid * jnp.int32(N_CHUNKS)
        pltpu.sync_copy(
            src_hbm.at[pl.ds(sid * jnp.int32(EDGES_PER_TILE),
                             EDGES_PER_TILE)], gidx)
        pltpu.sync_copy(dst_hbm.at[pl.ds(trow, N_CHUNKS)], didx)

        def fix_src(k, carry):
            sl = pl.ds(k * jnp.int32(16), 16)
            gidx[sl] = gidx[sl] * jnp.int32(2) + cid
            return carry

        lax.fori_loop(jnp.int32(0), jnp.int32(EDGES_PER_TILE // 16), fix_src,
                      jnp.int32(0))

        def fill_z(i, carry):
            for j in range(DH // 16):
                zbuf[i, pl.ds(j * 16, 16)] = jnp.zeros((16,), jnp.float32)
            zbufc[i, :] = jnp.zeros((16,), jnp.float32)
            return carry

        lax.fori_loop(jnp.int32(0), jnp.int32(ZROWS), fill_z, jnp.int32(0))

        def fill_ones(i, carry):
            ones_v[i, :] = jnp.ones((16,), jnp.float32)
            return carry

        lax.fori_loop(jnp.int32(0), jnp.int32(CHUNK), fill_ones, jnp.int32(0))

        for z in range(ROWS_PER_TILE // ZROWS):
            zr = r0 + jnp.int32(z * ZROWS)
            pltpu.sync_copy(zbuf, acc_sh.at[pl.ds(zr, ZROWS)])
            pltpu.sync_copy(zbufc, cnt_sh.at[pl.ds(zr, ZROWS)])
        plsc.subcore_barrier()

        def fire_gather(c, b):
            pltpu.async_copy(
                x_hbm.at[gidx.at[pl.ds(c * jnp.int32(CHUNK), CHUNK)]],
                rows.at[jnp.int32(b)], gsems[b])

        def my_cnt(c):
            return lax.rem(c, jnp.int32(2)) == cid

        def step(c, b):
            bl = (b + LA) % NBUF
            bi = jnp.int32(b)
            bli = jnp.int32(bl)

            @pl.when(c + jnp.int32(LA) < jnp.int32(N_CHUNKS))
            def _():
                @pl.when(c >= jnp.int32(1))
                def _():
                    pltpu.make_async_copy(
                        rows.at[bli], acc_sh.at[didx.at[c - jnp.int32(1)]],
                        ssems[bl]).wait()

                    @pl.when(my_cnt(c - jnp.int32(1)))
                    def _():
                        pltpu.make_async_copy(
                            ones_v, cnt_sh.at[didx.at[c - jnp.int32(1)]],
                            csems[bl]).wait()

                fire_gather(c + jnp.int32(LA), bl)

            pltpu.make_async_copy(
                x_hbm.at[gidx.at[pl.ds(c * jnp.int32(CHUNK), CHUNK)]],
                rows.at[bi], gsems[b]).wait()
            pltpu.async_copy(rows.at[bi], acc_sh.at[didx.at[c]], ssems[b],
                             add=True)

            @pl.when(my_cnt(c))
            def _():
                pltpu.async_copy(ones_v, cnt_sh.at[didx.at[c]], csems[b],
                                 add=True)

        for b in range(LA):
            fire_gather(jnp.int32(b), b)

        def quad_body(i, carry):
            cq = i * jnp.int32(NBUF)
            for b in range(NBUF):
                step(cq + jnp.int32(b), b)
            return carry

        lax.fori_loop(jnp.int32(0), jnp.int32(N_CHUNKS // NBUF), quad_body,
                      jnp.int32(0))

        for b in range(NBUF):
            cl = jnp.int32(N_CHUNKS - NBUF + b)
            pltpu.make_async_copy(rows.at[jnp.int32(b)],
                                  acc_sh.at[didx.at[cl]], ssems[b]).wait()

            @pl.when(my_cnt(cl))
            def _():
                pltpu.make_async_copy(ones_v, cnt_sh.at[didx.at[cl]],
                                      csems[b]).wait()

        plsc.subcore_barrier()

        out_r0 = cid * jnp.int32(N_PAD) + r0
        pltpu.sync_copy(acc_sh.at[pl.ds(r0, ROWS_PER_TILE)],
                        agg_out.at[pl.ds(out_r0, ROWS_PER_TILE)])
        pltpu.sync_copy(cnt_sh.at[pl.ds(r0, ROWS_PER_TILE)],
                        cnt_out.at[pl.ds(out_r0, ROWS_PER_TILE)])

    return agg_kernel(x01, src1d, dst2d)


def _tc_root_body(x_ref, wr_ref, b_ref, zr_ref):
    dims = (((1,), (1,)), ((), ()))
    zr_ref[...] = lax.dot_general(x_ref[...], wr_ref[...], dims,
                                  preferred_element_type=jnp.float32
                                  ) + b_ref[...]


def _tc_body(zr_ref, aggf_ref, cntf_ref, wl_ref, g_ref,
             be_ref, out_ref):
    agg = jnp.concatenate(
        [aggf_ref[pl.ds(0, N_NODES_C), :],
         aggf_ref[pl.ds(N_PAD, N_NODES_C), :]], axis=1)
    cnt = (cntf_ref[pl.ds(0, N_NODES_C), 0:1]
           + cntf_ref[pl.ds(N_PAD, N_NODES_C), 0:1])
    a = agg / jnp.maximum(cnt, 1.0)
    dims = (((1,), (1,)), ((), ()))
    z = lax.dot_general(a, wl_ref[...], dims,
                        preferred_element_type=jnp.float32)
    z = z + zr_ref[...]
    nrm = jnp.maximum(jnp.sqrt(jnp.sum(z * z, axis=1, keepdims=True)), 1e-12)
    h = jnp.maximum(z / nrm, 0.0)
    mean = jnp.mean(h, axis=0, keepdims=True)
    var = jnp.mean((h - mean) ** 2, axis=0, keepdims=True)
    out_ref[...] = (h - mean) * lax.rsqrt(var + 1e-5) * g_ref[...] + be_ref[...]


def kernel(x, edge_index, W_l, b_l, W_r, gamma, beta):
    x = x.astype(jnp.float32)
    src1d = edge_index[0].astype(jnp.int32)
    dst2d = edge_index[1].astype(jnp.int32).reshape(-1, CHUNK)
    x01 = x.reshape(2 * N_NODES_C, DH)

    agg_flat, cnt_flat = _sc_aggregate(x01, src1d, dst2d)

    zr = pl.pallas_call(
        _tc_root_body,
        out_shape=jax.ShapeDtypeStruct((N_NODES_C, D_C), jnp.float32),
    )(x, W_r.astype(jnp.float32), b_l.astype(jnp.float32).reshape(1, D_C))

    out = pl.pallas_call(
        _tc_body,
        out_shape=jax.ShapeDtypeStruct((N_NODES_C, D_C), jnp.float32),
    )(zr, agg_flat, cnt_flat,
      W_l.astype(jnp.float32),
      gamma.astype(jnp.float32).reshape(1, D_C),
      beta.astype(jnp.float32).reshape(1, D_C))
    return out


# speedup vs baseline: 1.0067x; 1.0067x over previous
import functools

import jax
import jax.numpy as jnp
from jax import lax
from jax.experimental import pallas as pl
from jax.experimental.pallas import tpu as pltpu
from jax.experimental.pallas import tpu_sc as plsc

N_NODES_C = 10000
N_PAD = 10240
N_EDGES_C = 320000
D_C = 128
DH = 64
CHUNK = 80
ROWS_PER_TILE = N_PAD // 16
EDGES_PER_TILE = N_EDGES_C // 16
N_CHUNKS = EDGES_PER_TILE // CHUNK
NBUF = 5
LA = NBUF - 1
ZROWS = 128


def _sc_aggregate(x01, src1d, dst2d):
    mesh = plsc.VectorSubcoreMesh(core_axis_name="c", subcore_axis_name="s")

    @functools.partial(
        pl.kernel,
        out_type=(
            jax.ShapeDtypeStruct((2 * N_PAD, DH), jnp.float32),
            jax.ShapeDtypeStruct((2 * N_PAD, 16), jnp.float32),
        ),
        mesh=mesh,
        compiler_params=pltpu.CompilerParams(use_tc_tiling_on_sc=False),
        scratch_types=[
            pltpu.VMEM_SHARED((N_PAD, DH), jnp.float32),
            pltpu.VMEM_SHARED((N_PAD, 16), jnp.float32),
            pltpu.VMEM((ZROWS, DH), jnp.float32),
            pltpu.VMEM((ZROWS, 16), jnp.float32),
            pltpu.VMEM((EDGES_PER_TILE,), jnp.int32),
            pltpu.VMEM((N_CHUNKS, CHUNK), jnp.int32),
            pltpu.VMEM((NBUF, CHUNK, DH), jnp.float32),
            pltpu.VMEM((CHUNK, 16), jnp.float32),
        ] + [pltpu.SemaphoreType.DMA] * (3 * NBUF),
    )
    def agg_kernel(x_hbm, src_hbm, dst_hbm, agg_out, cnt_out,
                   acc_sh, cnt_sh, zbuf, zbufc, gidx, didx, rows, ones_v,
                   *sems):
        cid = lax.axis_index("c")
        sid = lax.axis_index("s")
        r0 = sid * jnp.int32(ROWS_PER_TILE)
        gsems = sems[0:NBUF]
        ssems = sems[NBUF:2 * NBUF]
        csems = sems[2 * NBUF:3 * NBUF]

        trow = sid * jnp.int32(N_CHUNKS)
        pltpu.sync_copy(
            src_hbm.at[pl.ds(sid * jnp.int32(EDGES_PER_TILE),
                             EDGES_PER_TILE)], gidx)
        pltpu.sync_copy(dst_hbm.at[pl.ds(trow, N_CHUNKS)], didx)

        def fix_src(k, carry):
            sl = pl.ds(k * jnp.int32(16), 16)
            gidx[sl] = gidx[sl] * jnp.int32(2) + cid
            return carry

        lax.fori_loop(jnp.int32(0), jnp.int32(EDGES_PER_TILE // 16), fix_src,
                      jnp.int32(0))

        def fill_z(i, carry):
            for j in range(DH // 16):
                zbuf[i, pl.ds(j * 16, 16)] = jnp.zeros((16,), jnp.float32)
            zbufc[i, :] = jnp.zeros((16,), jnp.float32)
            return carry

        lax.fori_loop(jnp.int32(0), jnp.int32(ZROWS), fill_z, jnp.int32(0))

        def fill_ones(i, carry):
            ones_v[i, :] = jnp.ones((16,), jnp.float32)
            return carry

        lax.fori_loop(jnp.int32(0), jnp.int32(CHUNK), fill_ones, jnp.int32(0))

        for z in range(ROWS_PER_TILE // ZROWS):
            zr = r0 + jnp.int32(z * ZROWS)
            pltpu.sync_copy(zbuf, acc_sh.at[pl.ds(zr, ZROWS)])
            pltpu.sync_copy(zbufc, cnt_sh.at[pl.ds(zr, ZROWS)])
        plsc.subcore_barrier()

        def fire_gather(c, b):
            pltpu.async_copy(
                x_hbm.at[gidx.at[pl.ds(c * jnp.int32(CHUNK), CHUNK)]],
                rows.at[jnp.int32(b)], gsems[b])

        def my_cnt(c):
            return lax.rem(c, jnp.int32(2)) == cid

        def step(c, b):
            bl = (b + LA) % NBUF
            bi = jnp.int32(b)
            bli = jnp.int32(bl)

            @pl.when(c + jnp.int32(LA) < jnp.int32(N_CHUNKS))
            def _():
                @pl.when(c >= jnp.int32(1))
                def _():
                    pltpu.make_async_copy(
                        rows.at[bli], acc_sh.at[didx.at[c - jnp.int32(1)]],
                        ssems[bl]).wait()

                    @pl.when(my_cnt(c - jnp.int32(1)))
                    def _():
                        pltpu.make_async_copy(
                            ones_v, cnt_sh.at[didx.at[c - jnp.int32(1)]],
                            csems[bl]).wait()

                fire_gather(c + jnp.int32(LA), bl)

            pltpu.make_async_copy(
                x_hbm.at[gidx.at[pl.ds(c * jnp.int32(CHUNK), CHUNK)]],
                rows.at[bi], gsems[b]).wait()
            pltpu.async_copy(rows.at[bi], acc_sh.at[didx.at[c]], ssems[b],
                             add=True)

            @pl.when(my_cnt(c))
            def _():
                pltpu.async_copy(ones_v, cnt_sh.at[didx.at[c]], csems[b],
                                 add=True)

        for b in range(LA):
            fire_gather(jnp.int32(b), b)

        def quad_body(i, carry):
            cq = i * jnp.int32(NBUF)
            for b in range(NBUF):
                step(cq + jnp.int32(b), b)
            return carry

        lax.fori_loop(jnp.int32(0), jnp.int32(N_CHUNKS // NBUF), quad_body,
                      jnp.int32(0))

        for b in range(NBUF):
            cl = jnp.int32(N_CHUNKS - NBUF + b)
            pltpu.make_async_copy(rows.at[jnp.int32(b)],
                                  acc_sh.at[didx.at[cl]], ssems[b]).wait()

            @pl.when(my_cnt(cl))
            def _():
                pltpu.make_async_copy(ones_v, cnt_sh.at[didx.at[cl]],
                                      csems[b]).wait()

        plsc.subcore_barrier()

        out_r0 = cid * jnp.int32(N_PAD) + r0
        pltpu.sync_copy(acc_sh.at[pl.ds(r0, ROWS_PER_TILE)],
                        agg_out.at[pl.ds(out_r0, ROWS_PER_TILE)])
        pltpu.sync_copy(cnt_sh.at[pl.ds(r0, ROWS_PER_TILE)],
                        cnt_out.at[pl.ds(out_r0, ROWS_PER_TILE)])

    return agg_kernel(x01, src1d, dst2d)


def _tc_body(x_ref, aggf_ref, cntf_ref, wl_ref, wr_ref, b_ref, g_ref,
             be_ref, out_ref):
    agg = jnp.concatenate(
        [aggf_ref[pl.ds(0, N_NODES_C), :],
         aggf_ref[pl.ds(N_PAD, N_NODES_C), :]], axis=1)
    cnt = (cntf_ref[pl.ds(0, N_NODES_C), 0:1]
           + cntf_ref[pl.ds(N_PAD, N_NODES_C), 0:1])
    a = agg / jnp.maximum(cnt, 1.0)
    dims = (((1,), (1,)), ((), ()))
    z = lax.dot_general(a, wl_ref[...], dims,
                        preferred_element_type=jnp.float32)
    z = z + lax.dot_general(x_ref[...], wr_ref[...], dims,
                            preferred_element_type=jnp.float32)
    z = z + b_ref[...]
    nrm = jnp.maximum(jnp.sqrt(jnp.sum(z * z, axis=1, keepdims=True)), 1e-12)
    h = jnp.maximum(z / nrm, 0.0)
    mean = jnp.mean(h, axis=0, keepdims=True)
    var = jnp.mean((h - mean) ** 2, axis=0, keepdims=True)
    out_ref[...] = (h - mean) * lax.rsqrt(var + 1e-5) * g_ref[...] + be_ref[...]


def kernel(x, edge_index, W_l, b_l, W_r, gamma, beta):
    x = x.astype(jnp.float32)
    src1d = edge_index[0].astype(jnp.int32)
    dst2d = edge_index[1].astype(jnp.int32).reshape(-1, CHUNK)
    x01 = x.reshape(2 * N_NODES_C, DH)

    agg_flat, cnt_flat = _sc_aggregate(x01, src1d, dst2d)

    out = pl.pallas_call(
        _tc_body,
        out_shape=jax.ShapeDtypeStruct((N_NODES_C, D_C), jnp.float32),
    )(x, agg_flat, cnt_flat,
      W_l.astype(jnp.float32), W_r.astype(jnp.float32),
      b_l.astype(jnp.float32).reshape(1, D_C),
      gamma.astype(jnp.float32).reshape(1, D_C),
      beta.astype(jnp.float32).reshape(1, D_C))
    return out
